# baseline (speedup 1.0000x reference)
import jax
import jax.numpy as jnp
from jax import lax
from jax.experimental import pallas as pl
from jax.experimental.pallas import tpu as pltpu

_N, _C = 16384, 66
_NB = 64
_BLK = 2048
_SIGMA = 0.25
_THR = 0.001
_SP0, _SP1 = -100.0, -1000.0
_INV_SQRT2 = 0.7071067811865476


def _tc_body(x_ref, lab_ref, b_ref, out_ref):
    i = pl.program_id(0)
    x = x_ref[...]
    lab = lab_ref[...]
    b = b_ref[...]

    m = jnp.max(x, axis=1, keepdims=True)
    lse = jnp.log(jnp.sum(jnp.exp(x - m), axis=1, keepdims=True)) + m

    isp0 = (lab == _SP0).astype(jnp.float32)
    isp1 = (lab == _SP1).astype(jnp.float32)
    pad = isp0 + isp1

    z = (b - lab) * (_INV_SQRT2 / _SIGMA)
    cdf = 0.5 * (1.0 + lax.erf(z))
    p = cdf[:, 1:] - cdf[:, :-1]
    p = jnp.where(jnp.abs(p) >= _THR, p, 0.0)
    p = p * (1.0 - pad)

    s_mass = jnp.sum(p, axis=1, keepdims=True) + pad
    d_dot = (jnp.sum(p * x[:, 2:], axis=1, keepdims=True)
             + isp0 * x[:, 0:1] + isp1 * x[:, 1:2])
    part = jnp.sum(lse * s_mass - d_dot) * (1.0 / _N)

    @pl.when(i == 0)
    def _init():
        out_ref[0, 0] = 0.0

    out_ref[0, 0] += part


def kernel(inputs, labels, boundaries):
    grid = _N // _BLK
    out = pl.pallas_call(
        _tc_body,
        grid=(grid,),
        in_specs=[
            pl.BlockSpec((_BLK, _C), lambda i: (i, 0)),
            pl.BlockSpec((_BLK, 1), lambda i: (i, 0)),
            pl.BlockSpec((1, _NB + 1), lambda i: (0, 0)),
        ],
        out_specs=pl.BlockSpec(memory_space=pltpu.SMEM),
        out_shape=jax.ShapeDtypeStruct((1, 1), jnp.float32),
        compiler_params=pltpu.CompilerParams(
            dimension_semantics=("arbitrary",)),
    )(inputs, labels.reshape(_N, 1), boundaries.reshape(1, _NB + 1))
    return out[0, 0]


# speedup vs baseline: 2.7362x; 2.7362x over previous
import jax
import jax.numpy as jnp
from jax import lax
from jax.experimental import pallas as pl
from jax.experimental.pallas import tpu as pltpu

_N, _C = 16384, 66
_NB = 64
_BLKL = 2048
_SIGMA = 0.25
_THR = 0.001
_SP0, _SP1 = -100.0, -1000.0
_INV_SQRT2 = 0.7071067811865476


def _tc_body(xb_ref, xs_ref, lab_ref, b_ref, out_ref):
    i = pl.program_id(0)
    xb = xb_ref[...]
    xs = xs_ref[...]
    lab = lab_ref[...]
    b = b_ref[...]

    m = jnp.maximum(jnp.max(xb, axis=0, keepdims=True),
                    jnp.max(xs, axis=0, keepdims=True))
    se = (jnp.sum(jnp.exp(xb - m), axis=0, keepdims=True)
          + jnp.exp(xs[0:1, :] - m) + jnp.exp(xs[1:2, :] - m))
    lse = jnp.log(se) + m

    isp0 = (lab == _SP0).astype(jnp.float32)
    isp1 = (lab == _SP1).astype(jnp.float32)
    pad = isp0 + isp1

    z = (b - lab) * (_INV_SQRT2 / _SIGMA)
    cdf = 0.5 * (1.0 + lax.erf(z))
    p = cdf[1:, :] - cdf[:-1, :]
    p = jnp.where(jnp.abs(p) >= _THR, p, 0.0)
    p = p * (1.0 - pad)

    s_mass = jnp.sum(p, axis=0, keepdims=True) + pad
    d_dot = (jnp.sum(p * xb, axis=0, keepdims=True)
             + isp0 * xs[0:1, :] + isp1 * xs[1:2, :])
    part = jnp.sum(lse * s_mass - d_dot) * (1.0 / _N)

    @pl.when(i == 0)
    def _init():
        out_ref[0, 0] = 0.0

    out_ref[0, 0] += part


def kernel(inputs, labels, boundaries):
    xt = inputs.T
    xb = xt[2:, :]
    xs = xt[:2, :]
    grid = _N // _BLKL
    out = pl.pallas_call(
        _tc_body,
        grid=(grid,),
        in_specs=[
            pl.BlockSpec((_NB, _BLKL), lambda i: (0, i)),
            pl.BlockSpec((2, _BLKL), lambda i: (0, i)),
            pl.BlockSpec((1, _BLKL), lambda i: (0, i)),
            pl.BlockSpec((_NB + 1, 1), lambda i: (0, 0)),
        ],
        out_specs=pl.BlockSpec(memory_space=pltpu.SMEM),
        out_shape=jax.ShapeDtypeStruct((1, 1), jnp.float32),
        compiler_params=pltpu.CompilerParams(
            dimension_semantics=("arbitrary",)),
    )(xb, xs, labels.reshape(1, _N), boundaries.reshape(_NB + 1, 1))
    return out[0, 0]


# speedup vs baseline: 2.7493x; 1.0048x over previous
import jax
import jax.numpy as jnp
from jax import lax
from jax.experimental import pallas as pl
from jax.experimental.pallas import tpu as pltpu

_N, _C = 16384, 66
_NB = 64
_BLKL = 2048
_SIGMA = 0.25
_THR = 0.001
_SP0, _SP1 = -100.0, -1000.0
_INV_SQRT2 = 0.7071067811865476


def _tc_body(xb_ref, xs_ref, lab_ref, b_ref, out_ref):
    i = pl.program_id(0)
    xb = xb_ref[...]
    xs = xs_ref[...]
    lab = lab_ref[...]
    b = b_ref[...]

    m = jnp.maximum(jnp.max(xb, axis=0, keepdims=True),
                    jnp.max(xs, axis=0, keepdims=True))
    se = (jnp.sum(jnp.exp(xb - m), axis=0, keepdims=True)
          + jnp.exp(xs[0:1, :] - m) + jnp.exp(xs[1:2, :] - m))
    lse = jnp.log(se) + m

    isp0 = (lab == _SP0).astype(jnp.float32)
    isp1 = (lab == _SP1).astype(jnp.float32)
    pad = isp0 + isp1

    z = (b - lab) * (_INV_SQRT2 / _SIGMA)
    cdf = 0.5 * (1.0 + lax.erf(z))
    p = cdf[1:, :] - cdf[:-1, :]
    p = jnp.where(jnp.abs(p) >= _THR, p, 0.0)
    p = p * (1.0 - pad)

    s_mass = jnp.sum(p, axis=0, keepdims=True) + pad
    d_dot = (jnp.sum(p * xb, axis=0, keepdims=True)
             + isp0 * xs[0:1, :] + isp1 * xs[1:2, :])
    part = jnp.sum(lse * s_mass - d_dot) * (1.0 / _N)

    @pl.when(i == 0)
    def _init():
        out_ref[0, 0] = 0.0

    out_ref[0, 0] += part


def kernel(inputs, labels, boundaries):
    xb = inputs[:, 2:].T
    xs = inputs[:, :2].T
    grid = _N // _BLKL
    out = pl.pallas_call(
        _tc_body,
        grid=(grid,),
        in_specs=[
            pl.BlockSpec((_NB, _BLKL), lambda i: (0, i)),
            pl.BlockSpec((2, _BLKL), lambda i: (0, i)),
            pl.BlockSpec((1, _BLKL), lambda i: (0, i)),
            pl.BlockSpec((_NB + 1, 1), lambda i: (0, 0)),
        ],
        out_specs=pl.BlockSpec(memory_space=pltpu.SMEM),
        out_shape=jax.ShapeDtypeStruct((1, 1), jnp.float32),
        compiler_params=pltpu.CompilerParams(
            dimension_semantics=("arbitrary",)),
    )(xb, xs, labels.reshape(1, _N), boundaries.reshape(_NB + 1, 1))
    return out[0, 0]
